# baseline (speedup 1.0000x reference)
import functools
import math

import jax
import jax.numpy as jnp
from jax.experimental import pallas as pl

_MAX_RADIUS = 3.0
_NUM_BASIS = 3
_HID = 100
_STEP = _MAX_RADIUS / (_NUM_BASIS - 1)
_HIGHEST = jax.lax.Precision.HIGHEST
_ABLK = 48


def _conv_body(ga_ref, gb_ref, f_ref, wr1_ref, wr2_ref, o_ref, *, n_real):
    pa = ga_ref[0]
    pb = gb_ref[0]
    fb = f_ref[0]
    gram = jax.lax.dot_general(pa, pb, (((1,), (1,)), ((), ())),
                               precision=_HIGHEST)
    na = jnp.sum(pa * pa, axis=1)
    nb = jnp.sum(pb * pb, axis=1)
    d2 = na[:, None] + nb[None, :] - 2.0 * gram
    d = jnp.sqrt(jnp.maximum(d2, 0.0) + 1e-12)
    w1 = wr1_ref[...].astype(jnp.bfloat16).astype(jnp.float32)
    acc = None
    for i in range(_NUM_BASIS):
        u = (d - i * _STEP) / _STEP
        c = jnp.where(jnp.abs(u) < 1.0, jnp.cos(0.5 * math.pi * u) ** 2, 0.0)
        cb = c.astype(jnp.bfloat16).astype(jnp.float32)
        term = cb[:, :, None] * w1[i][None, None, :]
        acc = term if acc is None else acc + term
    h = jnp.maximum(acc / jnp.sqrt(jnp.float32(_NUM_BASIS)), 0.0)
    hb16 = h.astype(jnp.bfloat16).reshape(h.shape[0] * h.shape[1], _HID)
    w2b16 = wr2_ref[...].astype(jnp.bfloat16)
    k2d = jax.lax.dot_general(hb16, w2b16, (((1,), (0,)), ((), ())),
                              preferred_element_type=jnp.float32)
    kmat = (k2d / jnp.sqrt(jnp.float32(_HID))).reshape(h.shape[0], h.shape[1], -1)
    kb = kmat.astype(jnp.bfloat16).astype(jnp.float32)
    t = kb * fb[None, :, :]
    s = jnp.sum(jnp.sum(t, axis=1), axis=-1)
    o_ref[0, 0, 0] = s / jnp.sqrt(jnp.float32(n_real))


def _head_body(fn_ref, w1_ref, b1_ref, w2_ref, b2_ref, w3_ref, b3_ref, o_ref):
    h = jnp.dot(fn_ref[...], w1_ref[...], precision=_HIGHEST) + b1_ref[...]
    h = jnp.maximum(h, 0.0)
    h = jnp.dot(h, w2_ref[...], precision=_HIGHEST) + b2_ref[...]
    h = jnp.maximum(h, 0.0)
    o_ref[...] = jnp.dot(h, w3_ref[...], precision=_HIGHEST) + b3_ref[...]


def kernel(x, features, geometry, Wr1, Wr2, W1, b1, W2, b2, W3, b3):
    del x
    B, N, F = features.shape
    NP = -(-N // _ABLK) * _ABLK
    geo_p = jnp.pad(geometry, ((0, 0), (0, NP - N), (0, 0)))
    feat_p = jnp.pad(features, ((0, 0), (0, NP - N), (0, 0)))
    W1p = jnp.pad(W1, ((0, NP - N), (0, 0)))

    features_new = pl.pallas_call(
        functools.partial(_conv_body, n_real=N),
        grid=(B, NP // _ABLK),
        in_specs=[
            pl.BlockSpec((1, _ABLK, 3), lambda z, a: (z, a, 0)),
            pl.BlockSpec((1, NP, 3), lambda z, a: (z, 0, 0)),
            pl.BlockSpec((1, NP, F), lambda z, a: (z, 0, 0)),
            pl.BlockSpec((_NUM_BASIS, _HID), lambda z, a: (0, 0)),
            pl.BlockSpec((_HID, F), lambda z, a: (0, 0)),
        ],
        out_specs=pl.BlockSpec((1, 1, 1, _ABLK), lambda z, a: (z, a, 0, 0)),
        out_shape=jax.ShapeDtypeStruct((B, NP // _ABLK, 1, _ABLK), jnp.float32),
    )(geo_p, geo_p, feat_p, Wr1, Wr2)
    features_new = features_new.reshape(B, NP)

    b1r = b1.reshape(1, -1)
    b2r = b2.reshape(1, -1)
    b3r = b3.reshape(1, -1)
    out = pl.pallas_call(
        _head_body,
        out_shape=jax.ShapeDtypeStruct((B, 1), jnp.float32),
    )(features_new, W1p, b1r, W2, b2r, W3, b3r)
    return out


# speedup vs baseline: 1.9030x; 1.9030x over previous
import functools
import math

import jax
import jax.numpy as jnp
from jax.experimental import pallas as pl

_MAX_RADIUS = 3.0
_NUM_BASIS = 3
_HID = 100
_STEP = _MAX_RADIUS / (_NUM_BASIS - 1)
_HIGHEST = jax.lax.Precision.HIGHEST
_ABLK = 96


def _conv_body(ga_ref, gb_ref, f_ref, wr1_ref, wr2_ref, o_ref, *, n_real):
    pa = ga_ref[0]
    pb = gb_ref[0]
    fb = f_ref[0]
    gram = jax.lax.dot_general(pa, pb, (((1,), (1,)), ((), ())),
                               precision=_HIGHEST)
    na = jnp.sum(pa * pa, axis=1)
    nb = jnp.sum(pb * pb, axis=1)
    d2 = na[:, None] + nb[None, :] - 2.0 * gram
    d = jnp.sqrt(jnp.maximum(d2, 0.0) + 1e-12)
    w1b16 = wr1_ref[...].astype(jnp.bfloat16)
    v = d / _STEP
    cc = jnp.cos(0.5 * math.pi * v) ** 2
    zero = jnp.zeros_like(cc)
    cs = [
        jnp.where(v < 1.0, cc, zero).astype(jnp.bfloat16),
        jnp.where(jnp.abs(v - 1.0) < 1.0, 1.0 - cc, zero).astype(jnp.bfloat16),
        jnp.where(jnp.abs(v - 2.0) < 1.0, cc, zero).astype(jnp.bfloat16),
    ]
    c2d = jnp.stack(cs, axis=-1).reshape(d.shape[0] * d.shape[1], _NUM_BASIS)
    l2d = jax.lax.dot_general(c2d, w1b16, (((1,), (0,)), ((), ())),
                              preferred_element_type=jnp.float32)
    h2d = jnp.maximum(l2d / jnp.sqrt(jnp.float32(_NUM_BASIS)), 0.0)
    hb16 = h2d.astype(jnp.bfloat16)
    w2b16 = wr2_ref[...].astype(jnp.bfloat16)
    k2d = jax.lax.dot_general(hb16, w2b16, (((1,), (0,)), ((), ())),
                              preferred_element_type=jnp.float32)
    kmat = (k2d / jnp.sqrt(jnp.float32(_HID))).reshape(d.shape[0], d.shape[1], -1)
    kb = kmat.astype(jnp.bfloat16).astype(jnp.float32)
    t = kb * fb[None, :, :]
    s = jnp.sum(jnp.sum(t, axis=1), axis=-1)
    o_ref[0, 0, 0] = s / jnp.sqrt(jnp.float32(n_real))


def _head_body(fn_ref, w1_ref, b1_ref, w2_ref, b2_ref, w3_ref, b3_ref, o_ref):
    h = jnp.dot(fn_ref[...], w1_ref[...], precision=_HIGHEST) + b1_ref[...]
    h = jnp.maximum(h, 0.0)
    h = jnp.dot(h, w2_ref[...], precision=_HIGHEST) + b2_ref[...]
    h = jnp.maximum(h, 0.0)
    o_ref[...] = jnp.dot(h, w3_ref[...], precision=_HIGHEST) + b3_ref[...]


def kernel(x, features, geometry, Wr1, Wr2, W1, b1, W2, b2, W3, b3):
    del x
    B, N, F = features.shape
    NP = -(-N // _ABLK) * _ABLK
    geo_p = jnp.pad(geometry, ((0, 0), (0, NP - N), (0, 0)))
    feat_p = jnp.pad(features, ((0, 0), (0, NP - N), (0, 0)))
    W1p = jnp.pad(W1, ((0, NP - N), (0, 0)))

    features_new = pl.pallas_call(
        functools.partial(_conv_body, n_real=N),
        grid=(B, NP // _ABLK),
        in_specs=[
            pl.BlockSpec((1, _ABLK, 3), lambda z, a: (z, a, 0)),
            pl.BlockSpec((1, NP, 3), lambda z, a: (z, 0, 0)),
            pl.BlockSpec((1, NP, F), lambda z, a: (z, 0, 0)),
            pl.BlockSpec((_NUM_BASIS, _HID), lambda z, a: (0, 0)),
            pl.BlockSpec((_HID, F), lambda z, a: (0, 0)),
        ],
        out_specs=pl.BlockSpec((1, 1, 1, _ABLK), lambda z, a: (z, a, 0, 0)),
        out_shape=jax.ShapeDtypeStruct((B, NP // _ABLK, 1, _ABLK), jnp.float32),
    )(geo_p, geo_p, feat_p, Wr1, Wr2)
    features_new = features_new.reshape(B, NP)

    b1r = b1.reshape(1, -1)
    b2r = b2.reshape(1, -1)
    b3r = b3.reshape(1, -1)
    out = pl.pallas_call(
        _head_body,
        out_shape=jax.ShapeDtypeStruct((B, 1), jnp.float32),
    )(features_new, W1p, b1r, W2, b2r, W3, b3r)
    return out


# speedup vs baseline: 1.9273x; 1.0128x over previous
import functools
import math

import jax
import jax.numpy as jnp
from jax.experimental import pallas as pl

_MAX_RADIUS = 3.0
_NUM_BASIS = 3
_HID = 100
_STEP = _MAX_RADIUS / (_NUM_BASIS - 1)
_HIGHEST = jax.lax.Precision.HIGHEST
_ABLK = 96


def _conv_body(ga_ref, gb_ref, f_ref, wr1_ref, wr2_ref,
               w1_ref, b1_ref, w2_ref, b2_ref, w3_ref, b3_ref,
               o_ref, fn_ref, *, n_real):
    pa = ga_ref[0]
    pb = gb_ref[0]
    fb = f_ref[0]
    gram = jax.lax.dot_general(pa, pb, (((1,), (1,)), ((), ())),
                               precision=_HIGHEST)
    na = jnp.sum(pa * pa, axis=1)
    nb = jnp.sum(pb * pb, axis=1)
    d2 = na[:, None] + nb[None, :] - 2.0 * gram
    d = jnp.sqrt(jnp.maximum(d2, 0.0) + 1e-12)
    w1b16 = wr1_ref[...].astype(jnp.bfloat16)
    v = d / _STEP
    cc = jnp.cos(0.5 * math.pi * v) ** 2
    zero = jnp.zeros_like(cc)
    cs = [
        jnp.where(v < 1.0, cc, zero).astype(jnp.bfloat16),
        jnp.where(jnp.abs(v - 1.0) < 1.0, 1.0 - cc, zero).astype(jnp.bfloat16),
        jnp.where(jnp.abs(v - 2.0) < 1.0, cc, zero).astype(jnp.bfloat16),
    ]
    c2d = jnp.stack(cs, axis=-1).reshape(d.shape[0] * d.shape[1], _NUM_BASIS)
    l2d = jax.lax.dot_general(c2d, w1b16, (((1,), (0,)), ((), ())),
                              preferred_element_type=jnp.float32)
    h2d = jnp.maximum(l2d / jnp.sqrt(jnp.float32(_NUM_BASIS)), 0.0)
    hb16 = h2d.astype(jnp.bfloat16)
    w2b16 = wr2_ref[...].astype(jnp.bfloat16)
    k2d = jax.lax.dot_general(hb16, w2b16, (((1,), (0,)), ((), ())),
                              preferred_element_type=jnp.float32)
    kmat = (k2d / jnp.sqrt(jnp.float32(_HID))).reshape(d.shape[0], d.shape[1], -1)
    kb = kmat.astype(jnp.bfloat16).astype(jnp.float32)
    t = kb * fb[None, :, :]
    s = jnp.sum(jnp.sum(t, axis=1), axis=-1)
    z = pl.program_id(0)
    a = pl.program_id(1)
    fn_ref[z, a, 0, :] = s / jnp.sqrt(jnp.float32(n_real))

    @pl.when((z == pl.num_programs(0) - 1) & (a == pl.num_programs(1) - 1))
    def _head():
        fn = fn_ref[...].reshape(fn_ref.shape[0], -1)
        hh = jnp.dot(fn, w1_ref[...], precision=_HIGHEST) + b1_ref[...]
        hh = jnp.maximum(hh, 0.0)
        hh = jnp.dot(hh, w2_ref[...], precision=_HIGHEST) + b2_ref[...]
        hh = jnp.maximum(hh, 0.0)
        o_ref[...] = jnp.dot(hh, w3_ref[...], precision=_HIGHEST) + b3_ref[...]


def kernel(x, features, geometry, Wr1, Wr2, W1, b1, W2, b2, W3, b3):
    del x
    B, N, F = features.shape
    NP = -(-N // _ABLK) * _ABLK
    geo_p = jnp.pad(geometry, ((0, 0), (0, NP - N), (0, 0)))
    feat_p = jnp.pad(features, ((0, 0), (0, NP - N), (0, 0)))
    W1p = jnp.pad(W1, ((0, NP - N), (0, 0)))

    from jax.experimental.pallas import tpu as pltpu

    out = pl.pallas_call(
        functools.partial(_conv_body, n_real=N),
        grid=(B, NP // _ABLK),
        in_specs=[
            pl.BlockSpec((1, _ABLK, 3), lambda z, a: (z, a, 0)),
            pl.BlockSpec((1, NP, 3), lambda z, a: (z, 0, 0)),
            pl.BlockSpec((1, NP, F), lambda z, a: (z, 0, 0)),
            pl.BlockSpec((_NUM_BASIS, _HID), lambda z, a: (0, 0)),
            pl.BlockSpec((_HID, F), lambda z, a: (0, 0)),
            pl.BlockSpec((NP, 30), lambda z, a: (0, 0)),
            pl.BlockSpec((1, 30), lambda z, a: (0, 0)),
            pl.BlockSpec((30, 10), lambda z, a: (0, 0)),
            pl.BlockSpec((1, 10), lambda z, a: (0, 0)),
            pl.BlockSpec((10, 1), lambda z, a: (0, 0)),
            pl.BlockSpec((1, 1), lambda z, a: (0, 0)),
        ],
        out_specs=pl.BlockSpec((B, 1), lambda z, a: (0, 0)),
        out_shape=jax.ShapeDtypeStruct((B, 1), jnp.float32),
        scratch_shapes=[pltpu.VMEM((B, NP // _ABLK, 1, _ABLK), jnp.float32)],
    )(geo_p, geo_p, feat_p, Wr1, Wr2,
      W1p, b1.reshape(1, -1), W2, b2.reshape(1, -1), W3, b3.reshape(1, -1))
    return out
